# Initial kernel scaffold; baseline (speedup 1.0000x reference)
import jax
import jax.numpy as jnp
from jax.experimental import pallas as pl


def kernel(h, weight, mask, prototypes):
    raise NotImplementedError("write your pallas kernel here")


# speedup vs baseline: 1.7003x; 1.7003x over previous
import jax
import jax.numpy as jnp
from jax.experimental import pallas as pl
from jax.experimental.pallas import tpu as pltpu

HIDDEN = 1024
CB_INIT = 9
CB_PAD = 16
TAU = 0.07

BLOCK_M = 512


def _fused_kernel(h_ref, w_ref, m_ref, p_ref, z_ref, logits_ref):
    wm = w_ref[...] * m_ref[...]
    z = jax.lax.dot_general(
        h_ref[...], wm,
        dimension_numbers=(((1,), (1,)), ((), ())),
        preferred_element_type=jnp.float32,
    )
    z_ref[...] = z

    zn = z / jnp.maximum(
        jnp.sqrt(jnp.sum(z * z, axis=1, keepdims=True)), 1e-12)

    p = p_ref[...]
    pn = p / jnp.maximum(
        jnp.sqrt(jnp.sum(p * p, axis=1, keepdims=True)), 1e-12)

    logits_ref[...] = jax.lax.dot_general(
        zn, pn,
        dimension_numbers=(((1,), (1,)), ((), ())),
        preferred_element_type=jnp.float32,
    ) * (1.0 / TAU)


def kernel(h, weight, mask, prototypes):
    B, S, H = h.shape
    M = B * S
    hf = h.reshape(M, H)
    live = prototypes[:CB_INIT]
    p_pad = jnp.concatenate(
        [live, jnp.ones((CB_PAD - CB_INIT, H), live.dtype)], axis=0)

    grid = (M // BLOCK_M,)
    z, logits = pl.pallas_call(
        _fused_kernel,
        grid=grid,
        in_specs=[
            pl.BlockSpec((BLOCK_M, H), lambda i: (i, 0)),
            pl.BlockSpec((H, H), lambda i: (0, 0)),
            pl.BlockSpec((H, H), lambda i: (0, 0)),
            pl.BlockSpec((CB_PAD, H), lambda i: (0, 0)),
        ],
        out_specs=[
            pl.BlockSpec((BLOCK_M, H), lambda i: (i, 0)),
            pl.BlockSpec((BLOCK_M, CB_PAD), lambda i: (i, 0)),
        ],
        out_shape=[
            jax.ShapeDtypeStruct((M, H), jnp.float32),
            jax.ShapeDtypeStruct((M, CB_PAD), jnp.float32),
        ],
    )(hf, weight, mask, p_pad)

    return (logits[:, :CB_INIT].reshape(B, S, CB_INIT),
            z.reshape(B, S, H))
